# Initial kernel scaffold; baseline (speedup 1.0000x reference)
import functools

import jax
import jax.numpy as jnp
from jax import lax
from jax.experimental import pallas as pl
from jax.experimental.pallas import tpu as pltpu
from jax.experimental.pallas import tpu_sc as plsc

N = 10000
E = 160000
D = 256
DH = 128
NC = 2
NS = 16
EPT = E // NS
K = 80
RPT = N // NS

BN = 1000


def _lin_relu_body(x_ref, w_ref, b_ref, o_ref):
    h = lax.dot_general(x_ref[...], w_ref[...], (((1,), (1,)), ((), ())),
                        preferred_element_type=jnp.float32)
    r = jnp.maximum(h + b_ref[...], 0.0)
    o_ref[0] = r[:, :DH]
    o_ref[1] = r[:, DH:]


def _lin_relu(x, w_lin, b_lin):
    return pl.pallas_call(
        _lin_relu_body,
        grid=(N // BN,),
        in_specs=[
            pl.BlockSpec((BN, D), lambda i: (i, 0)),
            pl.BlockSpec((D, D), lambda i: (0, 0)),
            pl.BlockSpec((1, D), lambda i: (0, 0)),
        ],
        out_specs=pl.BlockSpec((NC, BN, DH), lambda i: (0, i, 0)),
        out_shape=jax.ShapeDtypeStruct((NC, N, DH), jnp.float32),
    )(x, w_lin, b_lin.reshape(1, D))


def _sc_scatter_body(r2, src2, dst, zeros, agg_out,
                     src_idx, dst_idx, rows, shared_agg, sem):
    c = lax.axis_index("c")
    s = lax.axis_index("s")

    pltpu.sync_copy(zeros.at[pl.ds(s * RPT, RPT)],
                    shared_agg.at[pl.ds(s * RPT, RPT)])
    plsc.subcore_barrier()

    def body(i, carry):
        base = s * EPT + i * K
        pltpu.sync_copy(src2.at[pl.ds(c * E + base, K)], src_idx)
        pltpu.sync_copy(dst.at[pl.ds(base, K)], dst_idx)
        pltpu.async_copy(r2.at[src_idx], rows, sem).wait()
        pltpu.sync_copy(rows, shared_agg.at[dst_idx], add=True)
        return carry

    lax.fori_loop(0, EPT // K, body, 0)
    plsc.subcore_barrier()

    pltpu.sync_copy(shared_agg.at[pl.ds(s * RPT, RPT)],
                    agg_out.at[pl.ds(c * N + s * RPT, RPT)])


def _sc_scatter(r2_flat, src2, dst, zeros):
    mesh = plsc.VectorSubcoreMesh(core_axis_name="c", subcore_axis_name="s")
    kfn = pl.kernel(
        _sc_scatter_body,
        out_type=jax.ShapeDtypeStruct((NC * N, DH), jnp.float32),
        mesh=mesh,
        scratch_types=[
            pltpu.VMEM((K,), jnp.int32),
            pltpu.VMEM((K,), jnp.int32),
            pltpu.VMEM((K, DH), jnp.float32),
            pltpu.VMEM_SHARED((N, DH), jnp.float32),
            pltpu.SemaphoreType.DMA,
        ],
    )
    return kfn(r2_flat, src2, dst, zeros)


def _gin_stats_body(x_ref, agg_ref, w1_ref, scale_ref, t_ref, st_ref):
    agg = jnp.concatenate([agg_ref[0], agg_ref[1]], axis=1)
    pre = x_ref[...] * scale_ref[0, 0] + agg
    t = lax.dot_general(pre, w1_ref[...], (((1,), (1,)), ((), ())),
                        preferred_element_type=jnp.float32)
    t_ref[...] = t

    @pl.when(pl.program_id(0) == 0)
    def _():
        st_ref[...] = jnp.zeros_like(st_ref)

    st_ref[0:1, :] += jnp.sum(t, axis=0, keepdims=True)
    st_ref[1:2, :] += jnp.sum(t * t, axis=0, keepdims=True)


def _gin_stats(x, agg2, w1, scale):
    return pl.pallas_call(
        _gin_stats_body,
        grid=(N // BN,),
        in_specs=[
            pl.BlockSpec((BN, D), lambda i: (i, 0)),
            pl.BlockSpec((NC, BN, DH), lambda i: (0, i, 0)),
            pl.BlockSpec((D, D), lambda i: (0, 0)),
            pl.BlockSpec((1, 1), lambda i: (0, 0)),
        ],
        out_specs=[
            pl.BlockSpec((BN, D), lambda i: (i, 0)),
            pl.BlockSpec((2, D), lambda i: (0, 0)),
        ],
        out_shape=[
            jax.ShapeDtypeStruct((N, D), jnp.float32),
            jax.ShapeDtypeStruct((2, D), jnp.float32),
        ],
    )(x, agg2, w1, scale)


def _bn_out_body(t_ref, st_ref, g_ref, b_ref, w2_ref, o_ref):
    inv_n = 1.0 / N
    mean = st_ref[0:1, :] * inv_n
    var = st_ref[1:2, :] * inv_n - mean * mean
    inv = lax.rsqrt(var + 1e-5)
    tn = (t_ref[...] - mean) * (inv * g_ref[...]) + b_ref[...]
    tn = jnp.maximum(tn, 0.0)
    o_ref[...] = lax.dot_general(tn, w2_ref[...], (((1,), (1,)), ((), ())),
                                 preferred_element_type=jnp.float32)


def _bn_out(t, stats, gamma, beta, w2):
    return pl.pallas_call(
        _bn_out_body,
        grid=(N // BN,),
        in_specs=[
            pl.BlockSpec((BN, D), lambda i: (i, 0)),
            pl.BlockSpec((2, D), lambda i: (0, 0)),
            pl.BlockSpec((1, D), lambda i: (0, 0)),
            pl.BlockSpec((1, D), lambda i: (0, 0)),
            pl.BlockSpec((D, D), lambda i: (0, 0)),
        ],
        out_specs=pl.BlockSpec((BN, D), lambda i: (i, 0)),
        out_shape=jax.ShapeDtypeStruct((N, D), jnp.float32),
    )(t, stats, gamma.reshape(1, D), beta.reshape(1, D), w2)


def kernel(x, edge_index, W_lin, b_lin, W1, gamma, beta, W2, eps_param):
    dst = edge_index[0].astype(jnp.int32)
    src = edge_index[1].astype(jnp.int32)
    src2 = jnp.concatenate([src, src + N])
    zeros = jnp.zeros((N, DH), jnp.float32)

    r2 = _lin_relu(x, W_lin, b_lin)
    agg_flat = _sc_scatter(r2.reshape(NC * N, DH), src2, dst, zeros)
    agg2 = agg_flat.reshape(NC, N, DH)

    scale = (1.0 + eps_param).reshape(1, 1)
    t, stats = _gin_stats(x, agg2, W1, scale)
    out = _bn_out(t, stats, gamma, beta, W2)
    return out


# speedup vs baseline: 3.5567x; 3.5567x over previous
import functools

import jax
import jax.numpy as jnp
from jax import lax
from jax.experimental import pallas as pl
from jax.experimental.pallas import tpu as pltpu
from jax.experimental.pallas import tpu_sc as plsc

N = 10000
E = 160000
D = 256
DH = 128
NC = 2
NS = 16
EPT = E // NS
K = 80
RPT = 624
RTAIL = N - NS * RPT

BN = 1000


def _lin_relu_body(x_ref, w_ref, b_ref, o_ref):
    h = lax.dot_general(x_ref[...], w_ref[...], (((1,), (1,)), ((), ())),
                        preferred_element_type=jnp.float32)
    r = jnp.maximum(h + b_ref[...], 0.0)
    o_ref[0] = r[:, :DH]
    o_ref[1] = r[:, DH:]


def _lin_relu(x, w_lin, b_lin):
    return pl.pallas_call(
        _lin_relu_body,
        grid=(N // BN,),
        in_specs=[
            pl.BlockSpec((BN, D), lambda i: (i, 0)),
            pl.BlockSpec((D, D), lambda i: (0, 0)),
            pl.BlockSpec((1, D), lambda i: (0, 0)),
        ],
        out_specs=pl.BlockSpec((NC, BN, DH), lambda i: (0, i, 0)),
        out_shape=jax.ShapeDtypeStruct((NC, N, DH), jnp.float32),
    )(x, w_lin, b_lin.reshape(1, D))


def _sc_scatter_body(r2, src2, dst, zeros, agg_out,
                     src_idx, dst_idx, rows, shared_agg, sem):
    c = lax.axis_index("c")
    s = lax.axis_index("s")

    pltpu.sync_copy(zeros.at[pl.ds(s * RPT, RPT)],
                    shared_agg.at[pl.ds(s * RPT, RPT)])

    @pl.when(s == NS - 1)
    def _():
        pltpu.sync_copy(zeros.at[pl.ds(NS * RPT, RTAIL)],
                        shared_agg.at[pl.ds(NS * RPT, RTAIL)])

    plsc.subcore_barrier()

    def body(i, carry):
        base = s * EPT + i * K
        pltpu.sync_copy(src2.at[pl.ds(c * E + base, K)], src_idx)
        pltpu.sync_copy(dst.at[pl.ds(base, K)], dst_idx)
        pltpu.async_copy(r2.at[src_idx], rows, sem).wait()
        pltpu.sync_copy(rows, shared_agg.at[dst_idx], add=True)
        return carry

    lax.fori_loop(0, EPT // K, body, 0)
    plsc.subcore_barrier()

    pltpu.sync_copy(shared_agg.at[pl.ds(s * RPT, RPT)],
                    agg_out.at[pl.ds(c * N + s * RPT, RPT)])

    @pl.when(s == NS - 1)
    def _():
        pltpu.sync_copy(shared_agg.at[pl.ds(NS * RPT, RTAIL)],
                        agg_out.at[pl.ds(c * N + NS * RPT, RTAIL)])


def _sc_scatter(r2_flat, src2, dst, zeros):
    mesh = plsc.VectorSubcoreMesh(core_axis_name="c", subcore_axis_name="s")
    kfn = pl.kernel(
        _sc_scatter_body,
        out_type=jax.ShapeDtypeStruct((NC * N, DH), jnp.float32),
        mesh=mesh,
        scratch_types=[
            pltpu.VMEM((K,), jnp.int32),
            pltpu.VMEM((K,), jnp.int32),
            pltpu.VMEM((K, DH), jnp.float32),
            pltpu.VMEM_SHARED((N, DH), jnp.float32),
            pltpu.SemaphoreType.DMA,
        ],
    )
    return kfn(r2_flat, src2, dst, zeros)


def _gin_stats_body(x_ref, agg_ref, w1_ref, scale_ref, t_ref, st_ref):
    agg = jnp.concatenate([agg_ref[0], agg_ref[1]], axis=1)
    pre = x_ref[...] * scale_ref[0, 0] + agg
    t = lax.dot_general(pre, w1_ref[...], (((1,), (1,)), ((), ())),
                        preferred_element_type=jnp.float32)
    t_ref[...] = t

    @pl.when(pl.program_id(0) == 0)
    def _():
        st_ref[...] = jnp.zeros_like(st_ref)

    st_ref[0:1, :] += jnp.sum(t, axis=0, keepdims=True)
    st_ref[1:2, :] += jnp.sum(t * t, axis=0, keepdims=True)


def _gin_stats(x, agg2, w1, scale):
    return pl.pallas_call(
        _gin_stats_body,
        grid=(N // BN,),
        in_specs=[
            pl.BlockSpec((BN, D), lambda i: (i, 0)),
            pl.BlockSpec((NC, BN, DH), lambda i: (0, i, 0)),
            pl.BlockSpec((D, D), lambda i: (0, 0)),
            pl.BlockSpec((1, 1), lambda i: (0, 0)),
        ],
        out_specs=[
            pl.BlockSpec((BN, D), lambda i: (i, 0)),
            pl.BlockSpec((2, D), lambda i: (0, 0)),
        ],
        out_shape=[
            jax.ShapeDtypeStruct((N, D), jnp.float32),
            jax.ShapeDtypeStruct((2, D), jnp.float32),
        ],
    )(x, agg2, w1, scale)


def _bn_out_body(t_ref, st_ref, g_ref, b_ref, w2_ref, o_ref):
    inv_n = 1.0 / N
    mean = st_ref[0:1, :] * inv_n
    var = st_ref[1:2, :] * inv_n - mean * mean
    inv = lax.rsqrt(var + 1e-5)
    tn = (t_ref[...] - mean) * (inv * g_ref[...]) + b_ref[...]
    tn = jnp.maximum(tn, 0.0)
    o_ref[...] = lax.dot_general(tn, w2_ref[...], (((1,), (1,)), ((), ())),
                                 preferred_element_type=jnp.float32)


def _bn_out(t, stats, gamma, beta, w2):
    return pl.pallas_call(
        _bn_out_body,
        grid=(N // BN,),
        in_specs=[
            pl.BlockSpec((BN, D), lambda i: (i, 0)),
            pl.BlockSpec((2, D), lambda i: (0, 0)),
            pl.BlockSpec((1, D), lambda i: (0, 0)),
            pl.BlockSpec((1, D), lambda i: (0, 0)),
            pl.BlockSpec((D, D), lambda i: (0, 0)),
        ],
        out_specs=pl.BlockSpec((BN, D), lambda i: (i, 0)),
        out_shape=jax.ShapeDtypeStruct((N, D), jnp.float32),
    )(t, stats, gamma.reshape(1, D), beta.reshape(1, D), w2)


def kernel(x, edge_index, W_lin, b_lin, W1, gamma, beta, W2, eps_param):
    dst = edge_index[0].astype(jnp.int32)
    src = edge_index[1].astype(jnp.int32)
    src2 = jnp.concatenate([src, src + N])
    zeros = jnp.zeros((N, DH), jnp.float32)

    r2 = _lin_relu(x, W_lin, b_lin)
    agg_flat = _sc_scatter(r2.reshape(NC * N, DH), src2, dst, zeros)
    agg2 = agg_flat.reshape(NC, N, DH)

    scale = (1.0 + eps_param).reshape(1, 1)
    t, stats = _gin_stats(x, agg2, W1, scale)
    out = _bn_out(t, stats, gamma, beta, W2)
    return out


# speedup vs baseline: 7.8800x; 2.2155x over previous
import jax
import jax.numpy as jnp
from jax import lax
from jax.experimental import pallas as pl
from jax.experimental.pallas import tpu as pltpu
from jax.experimental.pallas import tpu_sc as plsc

N = 10000
E = 160000
D = 256
DH = 128
NC = 2
NS = 16
EPT = E // NS
K = 80
RPT = 624
RTAIL = N - NS * RPT

BN = 1000


def _lin_relu_body(x_ref, w_ref, b_ref, o1_ref, o2_ref):
    h = lax.dot_general(x_ref[...], w_ref[...], (((1,), (1,)), ((), ())),
                        preferred_element_type=jnp.float32)
    r = jnp.maximum(h + b_ref[...], 0.0)
    o1_ref[...] = r[:, :DH]
    o2_ref[...] = r[:, DH:]


def _lin_relu(x, w_lin, b_lin):
    return pl.pallas_call(
        _lin_relu_body,
        grid=(N // BN,),
        in_specs=[
            pl.BlockSpec((BN, D), lambda i: (i, 0)),
            pl.BlockSpec((D, D), lambda i: (0, 0)),
            pl.BlockSpec((1, D), lambda i: (0, 0)),
        ],
        out_specs=[
            pl.BlockSpec((BN, DH), lambda i: (i, 0)),
            pl.BlockSpec((BN, DH), lambda i: (i, 0)),
        ],
        out_shape=[
            jax.ShapeDtypeStruct((N, DH), jnp.float32),
            jax.ShapeDtypeStruct((N, DH), jnp.float32),
        ],
    )(x, w_lin, b_lin.reshape(1, D))


CH = EPT // K
Q = 4
L = 2
QI = 8
LI = 5


def _sc_scatter_body(r2a, r2b, src, dst, zeros, agg0, agg1,
                     sidx, didx, rows, shared_agg, sisems, disems, gsems,
                     ssems):
    c = lax.axis_index("c")
    s = lax.axis_index("s")

    icps = {}
    gcps = {}
    scps = {}

    def issue_idx(i):
        b = i % QI
        base = s * EPT + i * K
        cp1 = pltpu.async_copy(src.at[pl.ds(base, K)], sidx[b], sisems[b])
        cp2 = pltpu.async_copy(dst.at[pl.ds(base, K)], didx[b], disems[b])
        icps[i] = (cp1, cp2)

    def issue_gather(i):
        b = i % Q
        icps[i][0].wait()

        @pl.when(c == 0)
        def _():
            gcps[i] = pltpu.async_copy(r2a.at[sidx[i % QI]], rows[b],
                                       gsems[b])

        @pl.when(c == 1)
        def _():
            pltpu.async_copy(r2b.at[sidx[i % QI]], rows[b], gsems[b])

    for j in range(min(LI, CH)):
        issue_idx(j)
    for j in range(min(L, CH)):
        issue_gather(j)

    pltpu.sync_copy(zeros.at[pl.ds(s * RPT, RPT)],
                    shared_agg.at[pl.ds(s * RPT, RPT)])

    @pl.when(s == NS - 1)
    def _():
        pltpu.sync_copy(zeros.at[pl.ds(NS * RPT, RTAIL)],
                        shared_agg.at[pl.ds(NS * RPT, RTAIL)])

    plsc.subcore_barrier()

    for i in range(CH):
        y = i + LI
        if y < CH:
            issue_idx(y)
        x = i + L
        if x < CH:
            if x - Q >= 0:
                scps[x - Q].wait()
            issue_gather(x)
        gcps[i].wait()
        icps[i][1].wait()
        scps[i] = pltpu.async_copy(rows[i % Q], shared_agg.at[didx[i % QI]],
                                   ssems[i % Q], add=True)

    for i in range(max(0, CH - Q), CH):
        scps[i].wait()

    plsc.subcore_barrier()

    def writeback(agg_out):
        pltpu.sync_copy(shared_agg.at[pl.ds(s * RPT, RPT)],
                        agg_out.at[pl.ds(s * RPT, RPT)])

        @pl.when(s == NS - 1)
        def _():
            pltpu.sync_copy(shared_agg.at[pl.ds(NS * RPT, RTAIL)],
                            agg_out.at[pl.ds(NS * RPT, RTAIL)])

    @pl.when(c == 0)
    def _():
        writeback(agg0)

    @pl.when(c == 1)
    def _():
        writeback(agg1)


def _sc_scatter(r2a, r2b, src, dst, zeros):
    mesh = plsc.VectorSubcoreMesh(core_axis_name="c", subcore_axis_name="s")
    kfn = pl.kernel(
        _sc_scatter_body,
        out_type=(
            jax.ShapeDtypeStruct((N, DH), jnp.float32),
            jax.ShapeDtypeStruct((N, DH), jnp.float32),
        ),
        mesh=mesh,
        scratch_types=[
            [pltpu.VMEM((K,), jnp.int32) for _ in range(QI)],
            [pltpu.VMEM((K,), jnp.int32) for _ in range(QI)],
            [pltpu.VMEM((K, DH), jnp.float32) for _ in range(Q)],
            pltpu.VMEM_SHARED((N, DH), jnp.float32),
            [pltpu.SemaphoreType.DMA for _ in range(QI)],
            [pltpu.SemaphoreType.DMA for _ in range(QI)],
            [pltpu.SemaphoreType.DMA for _ in range(Q)],
            [pltpu.SemaphoreType.DMA for _ in range(Q)],
        ],
    )
    return kfn(r2a, r2b, src, dst, zeros)


def _gin_stats_body(x_ref, a0_ref, a1_ref, w1_ref, scale_ref, t_ref, st_ref):
    agg = jnp.concatenate([a0_ref[...], a1_ref[...]], axis=1)
    pre = x_ref[...] * scale_ref[0, 0] + agg
    t = lax.dot_general(pre, w1_ref[...], (((1,), (1,)), ((), ())),
                        preferred_element_type=jnp.float32)
    t_ref[...] = t

    @pl.when(pl.program_id(0) == 0)
    def _():
        st_ref[...] = jnp.zeros_like(st_ref)

    st_ref[0:1, :] += jnp.sum(t, axis=0, keepdims=True)
    st_ref[1:2, :] += jnp.sum(t * t, axis=0, keepdims=True)


def _gin_stats(x, agg0, agg1, w1, scale):
    return pl.pallas_call(
        _gin_stats_body,
        grid=(N // BN,),
        in_specs=[
            pl.BlockSpec((BN, D), lambda i: (i, 0)),
            pl.BlockSpec((BN, DH), lambda i: (i, 0)),
            pl.BlockSpec((BN, DH), lambda i: (i, 0)),
            pl.BlockSpec((D, D), lambda i: (0, 0)),
            pl.BlockSpec((1, 1), lambda i: (0, 0)),
        ],
        out_specs=[
            pl.BlockSpec((BN, D), lambda i: (i, 0)),
            pl.BlockSpec((2, D), lambda i: (0, 0)),
        ],
        out_shape=[
            jax.ShapeDtypeStruct((N, D), jnp.float32),
            jax.ShapeDtypeStruct((2, D), jnp.float32),
        ],
    )(x, agg0, agg1, w1, scale)


def _bn_out_body(t_ref, st_ref, g_ref, b_ref, w2_ref, o_ref):
    inv_n = 1.0 / N
    mean = st_ref[0:1, :] * inv_n
    var = st_ref[1:2, :] * inv_n - mean * mean
    inv = lax.rsqrt(var + 1e-5)
    tn = (t_ref[...] - mean) * (inv * g_ref[...]) + b_ref[...]
    tn = jnp.maximum(tn, 0.0)
    o_ref[...] = lax.dot_general(tn, w2_ref[...], (((1,), (1,)), ((), ())),
                                 preferred_element_type=jnp.float32)


def _bn_out(t, stats, gamma, beta, w2):
    return pl.pallas_call(
        _bn_out_body,
        grid=(N // BN,),
        in_specs=[
            pl.BlockSpec((BN, D), lambda i: (i, 0)),
            pl.BlockSpec((2, D), lambda i: (0, 0)),
            pl.BlockSpec((1, D), lambda i: (0, 0)),
            pl.BlockSpec((1, D), lambda i: (0, 0)),
            pl.BlockSpec((D, D), lambda i: (0, 0)),
        ],
        out_specs=pl.BlockSpec((BN, D), lambda i: (i, 0)),
        out_shape=jax.ShapeDtypeStruct((N, D), jnp.float32),
    )(t, stats, gamma.reshape(1, D), beta.reshape(1, D), w2)


def kernel(x, edge_index, W_lin, b_lin, W1, gamma, beta, W2, eps_param):
    dst = edge_index[0].astype(jnp.int32)
    src = edge_index[1].astype(jnp.int32)
    zeros = jnp.zeros((N, DH), jnp.float32)

    r2a, r2b = _lin_relu(x, W_lin, b_lin)
    agg0, agg1 = _sc_scatter(r2a, r2b, src, dst, zeros)

    scale = (1.0 + eps_param).reshape(1, 1)
    t, stats = _gin_stats(x, agg0, agg1, W1, scale)
    out = _bn_out(t, stats, gamma, beta, W2)
    return out


# speedup vs baseline: 8.2417x; 1.0459x over previous
import jax
import jax.numpy as jnp
from jax import lax
from jax.experimental import pallas as pl
from jax.experimental.pallas import tpu as pltpu
from jax.experimental.pallas import tpu_sc as plsc

N = 10000
E = 160000
D = 256
DH = 128
NC = 2
NS = 16
EPT = E // NS
K = 80
RPT = 624
RTAIL = N - NS * RPT

BN = 1000


def _lin_relu_body(x_ref, w_ref, b_ref, o1_ref, o2_ref):
    h = lax.dot_general(x_ref[...], w_ref[...], (((1,), (1,)), ((), ())),
                        preferred_element_type=jnp.float32)
    r = jnp.maximum(h + b_ref[...], 0.0)
    o1_ref[...] = r[:, :DH]
    o2_ref[...] = r[:, DH:]


def _lin_relu(x, w_lin, b_lin):
    return pl.pallas_call(
        _lin_relu_body,
        grid=(N // BN,),
        in_specs=[
            pl.BlockSpec((BN, D), lambda i: (i, 0)),
            pl.BlockSpec((D, D), lambda i: (0, 0)),
            pl.BlockSpec((1, D), lambda i: (0, 0)),
        ],
        out_specs=[
            pl.BlockSpec((BN, DH), lambda i: (i, 0)),
            pl.BlockSpec((BN, DH), lambda i: (i, 0)),
        ],
        out_shape=[
            jax.ShapeDtypeStruct((N, DH), jnp.float32),
            jax.ShapeDtypeStruct((N, DH), jnp.float32),
        ],
    )(x, w_lin, b_lin.reshape(1, D))


CH = EPT // K
Q = 4
L = 2
QI = 8
LI = 5


def _sc_scatter_body(r2a, r2b, src, dst, zeros, agg0, agg1,
                     sidx, didx, rows, shared_agg, sisems, disems, gsems,
                     ssems):
    c = lax.axis_index("c")
    s = lax.axis_index("s")

    icps = {}
    gcps = {}
    scps = {}

    def issue_idx(i):
        b = i % QI
        base = s * EPT + i * K
        cp1 = pltpu.async_copy(src.at[pl.ds(base, K)], sidx[b], sisems[b])
        cp2 = pltpu.async_copy(dst.at[pl.ds(base, K)], didx[b], disems[b])
        icps[i] = (cp1, cp2)

    def issue_gather(i):
        b = i % Q
        icps[i][0].wait()

        @pl.when(c == 0)
        def _():
            gcps[i] = pltpu.async_copy(r2a.at[sidx[i % QI]], rows[b],
                                       gsems[b])

        @pl.when(c == 1)
        def _():
            pltpu.async_copy(r2b.at[sidx[i % QI]], rows[b], gsems[b])

    for j in range(min(LI, CH)):
        issue_idx(j)
    for j in range(min(L, CH)):
        issue_gather(j)

    pltpu.sync_copy(zeros.at[pl.ds(s * RPT, RPT)],
                    shared_agg.at[pl.ds(s * RPT, RPT)])

    @pl.when(s == NS - 1)
    def _():
        pltpu.sync_copy(zeros.at[pl.ds(NS * RPT, RTAIL)],
                        shared_agg.at[pl.ds(NS * RPT, RTAIL)])

    plsc.subcore_barrier()

    for i in range(CH):
        y = i + LI
        if y < CH:
            issue_idx(y)
        x = i + L
        if x < CH:
            if x - Q >= 0:
                scps[x - Q].wait()
            issue_gather(x)
        gcps[i].wait()
        icps[i][1].wait()
        scps[i] = pltpu.async_copy(rows[i % Q], shared_agg.at[didx[i % QI]],
                                   ssems[i % Q], add=True)

    for i in range(max(0, CH - Q), CH):
        scps[i].wait()

    plsc.subcore_barrier()

    def writeback(agg_out):
        pltpu.sync_copy(shared_agg.at[pl.ds(s * RPT, RPT)],
                        agg_out.at[pl.ds(s * RPT, RPT)])

        @pl.when(s == NS - 1)
        def _():
            pltpu.sync_copy(shared_agg.at[pl.ds(NS * RPT, RTAIL)],
                            agg_out.at[pl.ds(NS * RPT, RTAIL)])

    @pl.when(c == 0)
    def _():
        writeback(agg0)

    @pl.when(c == 1)
    def _():
        writeback(agg1)


def _sc_scatter(r2a, r2b, src, dst, zeros):
    mesh = plsc.VectorSubcoreMesh(core_axis_name="c", subcore_axis_name="s")
    kfn = pl.kernel(
        _sc_scatter_body,
        out_type=(
            jax.ShapeDtypeStruct((N, DH), jnp.float32),
            jax.ShapeDtypeStruct((N, DH), jnp.float32),
        ),
        mesh=mesh,
        scratch_types=[
            [pltpu.VMEM((K,), jnp.int32) for _ in range(QI)],
            [pltpu.VMEM((K,), jnp.int32) for _ in range(QI)],
            [pltpu.VMEM((K, DH), jnp.float32) for _ in range(Q)],
            pltpu.VMEM_SHARED((N, DH), jnp.float32),
            [pltpu.SemaphoreType.DMA for _ in range(QI)],
            [pltpu.SemaphoreType.DMA for _ in range(QI)],
            [pltpu.SemaphoreType.DMA for _ in range(Q)],
            [pltpu.SemaphoreType.DMA for _ in range(Q)],
        ],
    )
    return kfn(r2a, r2b, src, dst, zeros)


def _mlp_body(x_ref, a0_ref, a1_ref, w1_ref, scale_ref, g_ref, b_ref,
              w2_ref, o_ref, t_buf, st_ref):
    p = pl.program_id(0)
    i = pl.program_id(1)

    @pl.when(p == 0)
    def _():
        agg = jnp.concatenate([a0_ref[...], a1_ref[...]], axis=1)
        pre = x_ref[...] * scale_ref[0, 0] + agg
        t = lax.dot_general(pre, w1_ref[...], (((1,), (1,)), ((), ())),
                            preferred_element_type=jnp.float32)
        t_buf[pl.ds(i * BN, BN), :] = t

        @pl.when(i == 0)
        def _():
            st_ref[...] = jnp.zeros_like(st_ref)

        st_ref[0:1, :] += jnp.sum(t, axis=0, keepdims=True)
        st_ref[1:2, :] += jnp.sum(t * t, axis=0, keepdims=True)

    @pl.when(p == 1)
    def _():
        inv_n = 1.0 / N
        mean = st_ref[0:1, :] * inv_n
        var = st_ref[1:2, :] * inv_n - mean * mean
        inv = lax.rsqrt(var + 1e-5)
        t = t_buf[pl.ds(i * BN, BN), :]
        tn = (t - mean) * (inv * g_ref[...]) + b_ref[...]
        tn = jnp.maximum(tn, 0.0)
        o_ref[...] = lax.dot_general(tn, w2_ref[...],
                                     (((1,), (1,)), ((), ())),
                                     preferred_element_type=jnp.float32)


def _mlp(x, agg0, agg1, w1, scale, gamma, beta, w2):
    return pl.pallas_call(
        _mlp_body,
        grid=(2, N // BN),
        in_specs=[
            pl.BlockSpec((BN, D), lambda p, i: ((1 - p) * i, 0)),
            pl.BlockSpec((BN, DH), lambda p, i: ((1 - p) * i, 0)),
            pl.BlockSpec((BN, DH), lambda p, i: ((1 - p) * i, 0)),
            pl.BlockSpec((D, D), lambda p, i: (0, 0)),
            pl.BlockSpec((1, 1), lambda p, i: (0, 0)),
            pl.BlockSpec((1, D), lambda p, i: (0, 0)),
            pl.BlockSpec((1, D), lambda p, i: (0, 0)),
            pl.BlockSpec((D, D), lambda p, i: (0, 0)),
        ],
        out_specs=pl.BlockSpec((BN, D), lambda p, i: (p * i, 0)),
        out_shape=jax.ShapeDtypeStruct((N, D), jnp.float32),
        scratch_shapes=[
            pltpu.VMEM((N, D), jnp.float32),
            pltpu.VMEM((2, D), jnp.float32),
        ],
    )(x, agg0, agg1, w1, scale, gamma.reshape(1, D), beta.reshape(1, D), w2)


def kernel(x, edge_index, W_lin, b_lin, W1, gamma, beta, W2, eps_param):
    dst = edge_index[0].astype(jnp.int32)
    src = edge_index[1].astype(jnp.int32)
    zeros = jnp.zeros((N, DH), jnp.float32)

    r2a, r2b = _lin_relu(x, W_lin, b_lin)
    agg0, agg1 = _sc_scatter(r2a, r2b, src, dst, zeros)

    scale = (1.0 + eps_param).reshape(1, 1)
    out = _mlp(x, agg0, agg1, W1, scale, gamma, beta, W2)
    return out


# speedup vs baseline: 8.6850x; 1.0538x over previous
import jax
import jax.numpy as jnp
from jax import lax
from jax.experimental import pallas as pl
from jax.experimental.pallas import tpu as pltpu
from jax.experimental.pallas import tpu_sc as plsc

N = 10000
E = 160000
D = 256
DH = 128
NC = 2
NS = 16
EPT = E // NS
K = 80
RPT = 624
RTAIL = N - NS * RPT

BN = 1000


EB = 16384


def _lin_relu_body(x_ref, w_ref, b_ref, ei_ref, o1_ref, o2_ref, src_ref,
                   dst_ref):
    h = lax.dot_general(x_ref[...], w_ref[...], (((1,), (1,)), ((), ())),
                        preferred_element_type=jnp.float32)
    r = jnp.maximum(h + b_ref[...], 0.0)
    o1_ref[...] = r[:, :DH]
    o2_ref[...] = r[:, DH:]
    dst_ref[...] = ei_ref[0, :]
    src_ref[...] = ei_ref[1, :]


def _lin_relu(x, w_lin, b_lin, ei):
    return pl.pallas_call(
        _lin_relu_body,
        grid=(N // BN,),
        in_specs=[
            pl.BlockSpec((BN, D), lambda i: (i, 0)),
            pl.BlockSpec((D, D), lambda i: (0, 0)),
            pl.BlockSpec((1, D), lambda i: (0, 0)),
            pl.BlockSpec((2, EB), lambda i: (0, i)),
        ],
        out_specs=[
            pl.BlockSpec((BN, DH), lambda i: (i, 0)),
            pl.BlockSpec((BN, DH), lambda i: (i, 0)),
            pl.BlockSpec((EB,), lambda i: (i,)),
            pl.BlockSpec((EB,), lambda i: (i,)),
        ],
        out_shape=[
            jax.ShapeDtypeStruct((N, DH), jnp.float32),
            jax.ShapeDtypeStruct((N, DH), jnp.float32),
            jax.ShapeDtypeStruct((E,), jnp.int32),
            jax.ShapeDtypeStruct((E,), jnp.int32),
        ],
    )(x, w_lin, b_lin.reshape(1, D), ei)


CH = EPT // K
Q = 4
L = 2
QI = 8
LI = 5


def _sc_scatter_body(r2a, r2b, src, dst, zeros, agg0, agg1,
                     sidx, didx, rows, shared_agg, sisems, disems, gsems,
                     ssems):
    c = lax.axis_index("c")
    s = lax.axis_index("s")

    icps = {}
    gcps = {}
    scps = {}

    def issue_idx(i):
        b = i % QI
        base = s * EPT + i * K
        cp1 = pltpu.async_copy(src.at[pl.ds(base, K)], sidx[b], sisems[b])
        cp2 = pltpu.async_copy(dst.at[pl.ds(base, K)], didx[b], disems[b])
        icps[i] = (cp1, cp2)

    def issue_gather(i):
        b = i % Q
        icps[i][0].wait()

        @pl.when(c == 0)
        def _():
            gcps[i] = pltpu.async_copy(r2a.at[sidx[i % QI]], rows[b],
                                       gsems[b])

        @pl.when(c == 1)
        def _():
            pltpu.async_copy(r2b.at[sidx[i % QI]], rows[b], gsems[b])

    for j in range(min(LI, CH)):
        issue_idx(j)
    for j in range(min(L, CH)):
        issue_gather(j)

    pltpu.sync_copy(zeros.at[pl.ds(s * RPT, RPT)],
                    shared_agg.at[pl.ds(s * RPT, RPT)])

    @pl.when(s == NS - 1)
    def _():
        pltpu.sync_copy(zeros.at[pl.ds(NS * RPT, RTAIL)],
                        shared_agg.at[pl.ds(NS * RPT, RTAIL)])

    plsc.subcore_barrier()

    for i in range(CH):
        y = i + LI
        if y < CH:
            issue_idx(y)
        x = i + L
        if x < CH:
            if x - Q >= 0:
                scps[x - Q].wait()
            issue_gather(x)
        gcps[i].wait()
        icps[i][1].wait()
        scps[i] = pltpu.async_copy(rows[i % Q], shared_agg.at[didx[i % QI]],
                                   ssems[i % Q], add=True)

    for i in range(max(0, CH - Q), CH):
        scps[i].wait()

    plsc.subcore_barrier()

    def writeback(agg_out):
        pltpu.sync_copy(shared_agg.at[pl.ds(s * RPT, RPT)],
                        agg_out.at[pl.ds(s * RPT, RPT)])

        @pl.when(s == NS - 1)
        def _():
            pltpu.sync_copy(shared_agg.at[pl.ds(NS * RPT, RTAIL)],
                            agg_out.at[pl.ds(NS * RPT, RTAIL)])

    @pl.when(c == 0)
    def _():
        writeback(agg0)

    @pl.when(c == 1)
    def _():
        writeback(agg1)


def _sc_scatter(r2a, r2b, src, dst, zeros):
    mesh = plsc.VectorSubcoreMesh(core_axis_name="c", subcore_axis_name="s")
    kfn = pl.kernel(
        _sc_scatter_body,
        out_type=(
            jax.ShapeDtypeStruct((N, DH), jnp.float32),
            jax.ShapeDtypeStruct((N, DH), jnp.float32),
        ),
        mesh=mesh,
        scratch_types=[
            [pltpu.VMEM((K,), jnp.int32) for _ in range(QI)],
            [pltpu.VMEM((K,), jnp.int32) for _ in range(QI)],
            [pltpu.VMEM((K, DH), jnp.float32) for _ in range(Q)],
            pltpu.VMEM_SHARED((N, DH), jnp.float32),
            [pltpu.SemaphoreType.DMA for _ in range(QI)],
            [pltpu.SemaphoreType.DMA for _ in range(QI)],
            [pltpu.SemaphoreType.DMA for _ in range(Q)],
            [pltpu.SemaphoreType.DMA for _ in range(Q)],
        ],
    )
    return kfn(r2a, r2b, src, dst, zeros)


def _mlp_body(x_ref, a0_ref, a1_ref, w1_ref, scale_ref, g_ref, b_ref,
              w2_ref, o_ref, t_buf, st_ref):
    p = pl.program_id(0)
    i = pl.program_id(1)

    @pl.when(p == 0)
    def _():
        agg = jnp.concatenate([a0_ref[...], a1_ref[...]], axis=1)
        pre = x_ref[...] * scale_ref[0, 0] + agg
        t = lax.dot_general(pre, w1_ref[...], (((1,), (1,)), ((), ())),
                            preferred_element_type=jnp.float32)
        t_buf[pl.ds(i * BN, BN), :] = t

        @pl.when(i == 0)
        def _():
            st_ref[...] = jnp.zeros_like(st_ref)

        st_ref[0:1, :] += jnp.sum(t, axis=0, keepdims=True)
        st_ref[1:2, :] += jnp.sum(t * t, axis=0, keepdims=True)

    @pl.when(p == 1)
    def _():
        inv_n = 1.0 / N
        mean = st_ref[0:1, :] * inv_n
        var = st_ref[1:2, :] * inv_n - mean * mean
        inv = lax.rsqrt(var + 1e-5)
        t = t_buf[pl.ds(i * BN, BN), :]
        tn = (t - mean) * (inv * g_ref[...]) + b_ref[...]
        tn = jnp.maximum(tn, 0.0)
        o_ref[...] = lax.dot_general(tn, w2_ref[...],
                                     (((1,), (1,)), ((), ())),
                                     preferred_element_type=jnp.float32)


def _mlp(x, agg0, agg1, w1, scale, gamma, beta, w2):
    return pl.pallas_call(
        _mlp_body,
        grid=(2, N // BN),
        in_specs=[
            pl.BlockSpec((BN, D), lambda p, i: ((1 - p) * i, 0)),
            pl.BlockSpec((BN, DH), lambda p, i: ((1 - p) * i, 0)),
            pl.BlockSpec((BN, DH), lambda p, i: ((1 - p) * i, 0)),
            pl.BlockSpec((D, D), lambda p, i: (0, 0)),
            pl.BlockSpec((1, 1), lambda p, i: (0, 0)),
            pl.BlockSpec((1, D), lambda p, i: (0, 0)),
            pl.BlockSpec((1, D), lambda p, i: (0, 0)),
            pl.BlockSpec((D, D), lambda p, i: (0, 0)),
        ],
        out_specs=pl.BlockSpec((BN, D), lambda p, i: (p * i, 0)),
        out_shape=jax.ShapeDtypeStruct((N, D), jnp.float32),
        scratch_shapes=[
            pltpu.VMEM((N, D), jnp.float32),
            pltpu.VMEM((2, D), jnp.float32),
        ],
    )(x, agg0, agg1, w1, scale, gamma.reshape(1, D), beta.reshape(1, D), w2)


def kernel(x, edge_index, W_lin, b_lin, W1, gamma, beta, W2, eps_param):
    ei = edge_index.astype(jnp.int32)
    zeros = jnp.zeros((N, DH), jnp.float32)

    r2a, r2b, src, dst = _lin_relu(x, W_lin, b_lin, ei)
    agg0, agg1 = _sc_scatter(r2a, r2b, src, dst, zeros)

    scale = (1.0 + eps_param).reshape(1, 1)
    out = _mlp(x, agg0, agg1, W1, scale, gamma, beta, W2)
    return out


# speedup vs baseline: 8.7469x; 1.0071x over previous
import jax
import jax.numpy as jnp
from jax import lax
from jax.experimental import pallas as pl
from jax.experimental.pallas import tpu as pltpu
from jax.experimental.pallas import tpu_sc as plsc

N = 10000
E = 160000
D = 256
DH = 128
NC = 2
NS = 16
EPT = E // NS
K = 80
RPT = 624
RTAIL = N - NS * RPT

BN = 1000


EB = 16384


def _lin_relu_body(x_ref, w_ref, b_ref, ei_ref, o1_ref, o2_ref, src_ref,
                   dst_ref, z_ref):
    h = lax.dot_general(x_ref[...], w_ref[...], (((1,), (1,)), ((), ())),
                        preferred_element_type=jnp.float32)
    r = jnp.maximum(h + b_ref[...], 0.0)
    o1_ref[...] = r[:, :DH]
    o2_ref[...] = r[:, DH:]
    dst_ref[...] = ei_ref[0, :]
    src_ref[...] = ei_ref[1, :]
    z_ref[...] = jnp.zeros_like(z_ref)


def _lin_relu(x, w_lin, b_lin, ei):
    return pl.pallas_call(
        _lin_relu_body,
        grid=(N // BN,),
        in_specs=[
            pl.BlockSpec((BN, D), lambda i: (i, 0)),
            pl.BlockSpec((D, D), lambda i: (0, 0)),
            pl.BlockSpec((1, D), lambda i: (0, 0)),
            pl.BlockSpec((2, EB), lambda i: (0, i)),
        ],
        out_specs=[
            pl.BlockSpec((BN, DH), lambda i: (i, 0)),
            pl.BlockSpec((BN, DH), lambda i: (i, 0)),
            pl.BlockSpec((EB,), lambda i: (i,)),
            pl.BlockSpec((EB,), lambda i: (i,)),
            pl.BlockSpec((BN, DH), lambda i: (i, 0)),
        ],
        out_shape=[
            jax.ShapeDtypeStruct((N, DH), jnp.float32),
            jax.ShapeDtypeStruct((N, DH), jnp.float32),
            jax.ShapeDtypeStruct((E,), jnp.int32),
            jax.ShapeDtypeStruct((E,), jnp.int32),
            jax.ShapeDtypeStruct((N, DH), jnp.float32),
        ],
    )(x, w_lin, b_lin.reshape(1, D), ei)


CH = EPT // K
Q = 4
L = 2
QI = 8
LI = 5


def _sc_scatter_body(r2a, r2b, src, dst, zeros, agg0, agg1,
                     sidx, didx, rows, shared_agg, sisems, disems, gsems,
                     ssems):
    c = lax.axis_index("c")
    s = lax.axis_index("s")

    icps = {}
    gcps = {}
    scps = {}

    def issue_idx(i):
        b = i % QI
        base = s * EPT + i * K
        cp1 = pltpu.async_copy(src.at[pl.ds(base, K)], sidx[b], sisems[b])
        cp2 = pltpu.async_copy(dst.at[pl.ds(base, K)], didx[b], disems[b])
        icps[i] = (cp1, cp2)

    def issue_gather(i):
        b = i % Q
        icps[i][0].wait()

        @pl.when(c == 0)
        def _():
            gcps[i] = pltpu.async_copy(r2a.at[sidx[i % QI]], rows[b],
                                       gsems[b])

        @pl.when(c == 1)
        def _():
            pltpu.async_copy(r2b.at[sidx[i % QI]], rows[b], gsems[b])

    for j in range(min(LI, CH)):
        issue_idx(j)
    for j in range(min(L, CH)):
        issue_gather(j)

    pltpu.sync_copy(zeros.at[pl.ds(s * RPT, RPT)],
                    shared_agg.at[pl.ds(s * RPT, RPT)])

    @pl.when(s == NS - 1)
    def _():
        pltpu.sync_copy(zeros.at[pl.ds(NS * RPT, RTAIL)],
                        shared_agg.at[pl.ds(NS * RPT, RTAIL)])

    plsc.subcore_barrier()

    for i in range(CH):
        y = i + LI
        if y < CH:
            issue_idx(y)
        x = i + L
        if x < CH:
            if x - Q >= 0:
                scps[x - Q].wait()
            issue_gather(x)
        gcps[i].wait()
        icps[i][1].wait()
        scps[i] = pltpu.async_copy(rows[i % Q], shared_agg.at[didx[i % QI]],
                                   ssems[i % Q], add=True)

    for i in range(max(0, CH - Q), CH):
        scps[i].wait()

    plsc.subcore_barrier()

    def writeback(agg_out):
        pltpu.sync_copy(shared_agg.at[pl.ds(s * RPT, RPT)],
                        agg_out.at[pl.ds(s * RPT, RPT)])

        @pl.when(s == NS - 1)
        def _():
            pltpu.sync_copy(shared_agg.at[pl.ds(NS * RPT, RTAIL)],
                            agg_out.at[pl.ds(NS * RPT, RTAIL)])

    @pl.when(c == 0)
    def _():
        writeback(agg0)

    @pl.when(c == 1)
    def _():
        writeback(agg1)


def _sc_scatter(r2a, r2b, src, dst, zeros):
    mesh = plsc.VectorSubcoreMesh(core_axis_name="c", subcore_axis_name="s")
    kfn = pl.kernel(
        _sc_scatter_body,
        out_type=(
            jax.ShapeDtypeStruct((N, DH), jnp.float32),
            jax.ShapeDtypeStruct((N, DH), jnp.float32),
        ),
        mesh=mesh,
        scratch_types=[
            [pltpu.VMEM((K,), jnp.int32) for _ in range(QI)],
            [pltpu.VMEM((K,), jnp.int32) for _ in range(QI)],
            [pltpu.VMEM((K, DH), jnp.float32) for _ in range(Q)],
            pltpu.VMEM_SHARED((N, DH), jnp.float32),
            [pltpu.SemaphoreType.DMA for _ in range(QI)],
            [pltpu.SemaphoreType.DMA for _ in range(QI)],
            [pltpu.SemaphoreType.DMA for _ in range(Q)],
            [pltpu.SemaphoreType.DMA for _ in range(Q)],
        ],
    )
    return kfn(r2a, r2b, src, dst, zeros)


def _mlp_body(x_ref, a0_ref, a1_ref, w1_ref, scale_ref, g_ref, b_ref,
              w2_ref, o_ref, t_buf, st_ref):
    p = pl.program_id(0)
    i = pl.program_id(1)

    @pl.when(p == 0)
    def _():
        agg = jnp.concatenate([a0_ref[...], a1_ref[...]], axis=1)
        pre = x_ref[...] * scale_ref[0, 0] + agg
        t = lax.dot_general(pre, w1_ref[...], (((1,), (1,)), ((), ())),
                            preferred_element_type=jnp.float32)
        t_buf[pl.ds(i * BN, BN), :] = t

        @pl.when(i == 0)
        def _():
            st_ref[...] = jnp.zeros_like(st_ref)

        st_ref[0:1, :] += jnp.sum(t, axis=0, keepdims=True)
        st_ref[1:2, :] += jnp.sum(t * t, axis=0, keepdims=True)

    @pl.when(p == 1)
    def _():
        inv_n = 1.0 / N
        mean = st_ref[0:1, :] * inv_n
        var = st_ref[1:2, :] * inv_n - mean * mean
        inv = lax.rsqrt(var + 1e-5)
        t = t_buf[pl.ds(i * BN, BN), :]
        tn = (t - mean) * (inv * g_ref[...]) + b_ref[...]
        tn = jnp.maximum(tn, 0.0)
        o_ref[...] = lax.dot_general(tn, w2_ref[...],
                                     (((1,), (1,)), ((), ())),
                                     preferred_element_type=jnp.float32)


def _mlp(x, agg0, agg1, w1, scale, gamma, beta, w2):
    return pl.pallas_call(
        _mlp_body,
        grid=(2, N // BN),
        in_specs=[
            pl.BlockSpec((BN, D), lambda p, i: ((1 - p) * i, 0)),
            pl.BlockSpec((BN, DH), lambda p, i: ((1 - p) * i, 0)),
            pl.BlockSpec((BN, DH), lambda p, i: ((1 - p) * i, 0)),
            pl.BlockSpec((D, D), lambda p, i: (0, 0)),
            pl.BlockSpec((1, 1), lambda p, i: (0, 0)),
            pl.BlockSpec((1, D), lambda p, i: (0, 0)),
            pl.BlockSpec((1, D), lambda p, i: (0, 0)),
            pl.BlockSpec((D, D), lambda p, i: (0, 0)),
        ],
        out_specs=pl.BlockSpec((BN, D), lambda p, i: (p * i, 0)),
        out_shape=jax.ShapeDtypeStruct((N, D), jnp.float32),
        scratch_shapes=[
            pltpu.VMEM((N, D), jnp.float32),
            pltpu.VMEM((2, D), jnp.float32),
        ],
    )(x, agg0, agg1, w1, scale, gamma.reshape(1, D), beta.reshape(1, D), w2)


def kernel(x, edge_index, W_lin, b_lin, W1, gamma, beta, W2, eps_param):
    ei = edge_index.astype(jnp.int32)

    r2a, r2b, src, dst, zeros = _lin_relu(x, W_lin, b_lin, ei)
    agg0, agg1 = _sc_scatter(r2a, r2b, src, dst, zeros)

    scale = (1.0 + eps_param).reshape(1, 1)
    out = _mlp(x, agg0, agg1, W1, scale, gamma, beta, W2)
    return out
